# baseline (speedup 1.0000x reference)
import functools

import jax
import jax.numpy as jnp
from jax import lax
from jax.experimental import pallas as pl
from jax.experimental.pallas import tpu as pltpu
from jax.experimental.pallas import tpu_sc as plsc

_NW = 32
_CH = 32
_LANES = 16
_BS = 1024
_SC_BATCH = 1


def _sc_kernel_body(x_hbm, pe_hbm, out_hbm, pe_v, xa, xb, in_a, in_b, out_a, out_b):
    seq = x_hbm.shape[1]
    dim = x_hbm.shape[2]
    b0 = x_hbm.shape[0] - _SC_BATCH
    rows_per_w = seq // _NW
    n_chunks = rows_per_w // _CH
    vec_per_row = dim // _LANES
    n_steps = n_chunks * _SC_BATCH

    cid = lax.axis_index("c")
    sid = lax.axis_index("s")
    wid = sid * 2 + cid
    base = wid * rows_per_w

    bufs = (xa, xb)
    in_sems = (in_a, in_b)
    out_sems = (out_a, out_b)

    def row0_of(t):
        return base + (t // _SC_BATCH) * _CH

    def start_load(t, buf, sem):
        pltpu.async_copy(x_hbm.at[b0 + t % _SC_BATCH, pl.ds(row0_of(t), _CH)], buf, sem)

    def wait_load(t, buf, sem):
        pltpu.make_async_copy(
            x_hbm.at[b0 + t % _SC_BATCH, pl.ds(row0_of(t), _CH)], buf, sem).wait()

    def start_store(t, buf, sem):
        pltpu.async_copy(buf, out_hbm.at[t % _SC_BATCH, pl.ds(row0_of(t), _CH)], sem)

    def wait_store(t, buf, sem):
        pltpu.make_async_copy(
            buf, out_hbm.at[t % _SC_BATCH, pl.ds(row0_of(t), _CH)], sem).wait()

    start_load(0, bufs[0], in_sems[0])

    def pair_body(p, _):
        for k in range(2):
            t = p * 2 + k
            cur, nxt = bufs[k], bufs[1 - k]

            @pl.when(t % _SC_BATCH == 0)
            def _():
                pltpu.sync_copy(pe_hbm.at[pl.ds(row0_of(t), _CH)], pe_v)

            @pl.when(jnp.logical_and(t >= 1, t + 1 < n_steps))
            def _():
                wait_store(t - 1, nxt, out_sems[1 - k])

            @pl.when(t + 1 < n_steps)
            def _():
                start_load(t + 1, nxt, in_sems[1 - k])

            wait_load(t, cur, in_sems[k])

            def row_body(r, _):
                for j in range(vec_per_row):
                    sl = pl.ds(j * _LANES, _LANES)
                    xx = cur[r, sl]
                    pp = pe_v[r, sl]
                    cur[r, sl] = jnp.where(xx == 0.0, xx, xx + pp)
                return 0

            lax.fori_loop(0, _CH, row_body, 0)
            start_store(t, cur, out_sems[k])
        return 0

    lax.fori_loop(0, n_steps // 2, pair_body, 0)

    wait_store(n_steps - 2, bufs[0], out_sems[0])
    wait_store(n_steps - 1, bufs[1], out_sems[1])


def _sc_part(x, pe):
    batch, seq, dim = x.shape
    mesh = plsc.VectorSubcoreMesh(core_axis_name="c", subcore_axis_name="s")
    k = functools.partial(
        pl.kernel,
        mesh=mesh,
        out_type=jax.ShapeDtypeStruct((_SC_BATCH, seq, dim), x.dtype),
        scratch_types=[
            pltpu.VMEM((_CH, dim), jnp.float32),
            pltpu.VMEM((_CH, dim), jnp.float32),
            pltpu.VMEM((_CH, dim), jnp.float32),
            pltpu.SemaphoreType.DMA,
            pltpu.SemaphoreType.DMA,
            pltpu.SemaphoreType.DMA,
            pltpu.SemaphoreType.DMA,
        ],
    )(_sc_kernel_body)
    return k(x, pe)


def _pe_add_kernel(x_ref, pe_ref, out_ref):
    x = x_ref[...]
    pe = pe_ref[...]
    out_ref[...] = jnp.where(x == 0.0, x, x + pe[None, :, :])


def _tc_part(x, pe):
    batch, seq, dim = x.shape
    nb = batch - _SC_BATCH
    grid = (seq // _BS,)
    return pl.pallas_call(
        _pe_add_kernel,
        grid=grid,
        in_specs=[
            pl.BlockSpec((nb, _BS, dim), lambda s: (0, s, 0)),
            pl.BlockSpec((_BS, dim), lambda s: (s, 0)),
        ],
        out_specs=pl.BlockSpec((nb, _BS, dim), lambda s: (0, s, 0)),
        out_shape=jax.ShapeDtypeStruct((nb, seq, dim), x.dtype),
    )(x, pe)


def kernel(x, pos_embed):
    batch, seq, dim = x.shape
    pe = pos_embed[:seq]
    tc_out = _tc_part(x, pe)
    sc_out = _sc_part(x, pe)
    return jnp.concatenate([tc_out, sc_out], axis=0)


# speedup vs baseline: 1.3639x; 1.3639x over previous
import functools

import jax
import jax.numpy as jnp
from jax import lax
from jax.experimental import pallas as pl
from jax.experimental.pallas import tpu as pltpu
from jax.experimental.pallas import tpu_sc as plsc

_NW = 32
_CH = 32
_LANES = 16


def _sc_kernel_body(x_hbm, pe_hbm, out_hbm,
                    pe0, pe1, xa, xb,
                    pe_s0, pe_s1, in_a, in_b, out_a, out_b):
    batch = x_hbm.shape[0]
    seq = x_hbm.shape[1]
    dim = x_hbm.shape[2]
    rows_per_w = seq // _NW
    n_chunks = rows_per_w // _CH
    vec_per_row = dim // _LANES

    cid = lax.axis_index("c")
    sid = lax.axis_index("s")
    wid = sid * 2 + cid
    base = wid * rows_per_w

    pe_bufs = (pe0, pe1)
    pe_sems = (pe_s0, pe_s1)
    x_bufs = (xa, xb)
    in_sems = (in_a, in_b)
    out_sems = (out_a, out_b)

    def start_pe(c, buf, sem):
        pltpu.async_copy(pe_hbm.at[pl.ds(base + c * _CH, _CH)], buf, sem)

    def wait_pe(c, buf, sem):
        pltpu.make_async_copy(pe_hbm.at[pl.ds(base + c * _CH, _CH)], buf, sem).wait()

    def start_load(t, buf, sem):
        pltpu.async_copy(
            x_hbm.at[t % batch, pl.ds(base + (t // batch) * _CH, _CH)], buf, sem)

    def wait_load(t, buf, sem):
        pltpu.make_async_copy(
            x_hbm.at[t % batch, pl.ds(base + (t // batch) * _CH, _CH)], buf, sem).wait()

    def start_store(t, buf, sem):
        pltpu.async_copy(
            buf, out_hbm.at[t % batch, pl.ds(base + (t // batch) * _CH, _CH)], sem)

    def wait_store(t, buf, sem):
        pltpu.make_async_copy(
            buf, out_hbm.at[t % batch, pl.ds(base + (t // batch) * _CH, _CH)], sem).wait()

    n_steps = n_chunks * batch

    start_pe(0, pe_bufs[0], pe_sems[0])
    start_load(0, x_bufs[0], in_sems[0])

    def pair_body(p, _):
        for kc in range(2):
            c = p * 2 + kc
            pe_cur = pe_bufs[kc]

            for b in range(batch):
                t_static = kc * batch + b
                t = p * (2 * batch) + t_static
                kx = t_static % 2
                cur, nxt = x_bufs[kx], x_bufs[1 - kx]

                @pl.when(jnp.logical_and(t >= 1, t + 1 < n_steps))
                def _():
                    wait_store(t - 1, nxt, out_sems[1 - kx])

                @pl.when(t + 1 < n_steps)
                def _():
                    start_load(t + 1, nxt, in_sems[1 - kx])

                if b == 0:
                    wait_pe(c, pe_cur, pe_sems[kc])

                    @pl.when(c + 1 < n_chunks)
                    def _():
                        start_pe(c + 1, pe_bufs[1 - kc], pe_sems[1 - kc])

                wait_load(t, cur, in_sems[kx])

                def row_body(r, _):
                    for j in range(vec_per_row):
                        sl = pl.ds(j * _LANES, _LANES)
                        xx = cur[r, sl]
                        pp = pe_cur[r, sl]
                        cur[r, sl] = jnp.where(xx == 0.0, xx, xx + pp)
                    return 0

                lax.fori_loop(0, _CH, row_body, 0)
                start_store(t, cur, out_sems[kx])
        return 0

    lax.fori_loop(0, n_chunks // 2, pair_body, 0)

    wait_store(n_steps - 2, x_bufs[(n_steps - 2) % 2], out_sems[(n_steps - 2) % 2])
    wait_store(n_steps - 1, x_bufs[(n_steps - 1) % 2], out_sems[(n_steps - 1) % 2])


def kernel(x, pos_embed):
    batch, seq, dim = x.shape
    pe = pos_embed[:seq]
    mesh = plsc.VectorSubcoreMesh(core_axis_name="c", subcore_axis_name="s")
    k = functools.partial(
        pl.kernel,
        mesh=mesh,
        out_type=jax.ShapeDtypeStruct(x.shape, x.dtype),
        scratch_types=[
            pltpu.VMEM((_CH, dim), jnp.float32),
            pltpu.VMEM((_CH, dim), jnp.float32),
            pltpu.VMEM((_CH, dim), jnp.float32),
            pltpu.VMEM((_CH, dim), jnp.float32),
            pltpu.SemaphoreType.DMA,
            pltpu.SemaphoreType.DMA,
            pltpu.SemaphoreType.DMA,
            pltpu.SemaphoreType.DMA,
            pltpu.SemaphoreType.DMA,
            pltpu.SemaphoreType.DMA,
        ],
    )(_sc_kernel_body)
    return k(x, pe)


# speedup vs baseline: 1.4205x; 1.0415x over previous
import functools

import jax
import jax.numpy as jnp
from jax import lax
from jax.experimental import pallas as pl
from jax.experimental.pallas import tpu as pltpu
from jax.experimental.pallas import tpu_sc as plsc

_NW = 32
_CH = 16
_LANES = 16


def _sc_kernel_body(x_hbm, pe_hbm, out_hbm,
                    pe0, pe1, x00, x01, x02, x03, x10, x11, x12, x13,
                    pe_s0, pe_s1, in_s0, in_s1, out_s0, out_s1):
    batch = x_hbm.shape[0]
    seq = x_hbm.shape[1]
    dim = x_hbm.shape[2]
    rows_per_w = seq // _NW
    n_chunks = rows_per_w // _CH
    vec_per_row = dim // _LANES

    cid = lax.axis_index("c")
    sid = lax.axis_index("s")
    wid = sid * 2 + cid
    base = wid * rows_per_w

    pe_bufs = (pe0, pe1)
    pe_sems = (pe_s0, pe_s1)
    x_sets = ((x00, x01, x02, x03), (x10, x11, x12, x13))
    in_sems = (in_s0, in_s1)
    out_sems = (out_s0, out_s1)

    def start_pe(c, buf, sem):
        pltpu.async_copy(pe_hbm.at[pl.ds(base + c * _CH, _CH)], buf, sem)

    def wait_pe(c, buf, sem):
        pltpu.make_async_copy(pe_hbm.at[pl.ds(base + c * _CH, _CH)], buf, sem).wait()

    def start_loads(c, bufs, sem):
        for b in range(batch):
            pltpu.async_copy(x_hbm.at[b, pl.ds(base + c * _CH, _CH)], bufs[b], sem)

    def wait_loads(c, bufs, sem):
        for b in range(batch):
            pltpu.make_async_copy(
                x_hbm.at[b, pl.ds(base + c * _CH, _CH)], bufs[b], sem).wait()

    def start_stores(c, bufs, sem):
        for b in range(batch):
            pltpu.async_copy(bufs[b], out_hbm.at[b, pl.ds(base + c * _CH, _CH)], sem)

    def wait_stores(c, bufs, sem):
        for b in range(batch):
            pltpu.make_async_copy(
                bufs[b], out_hbm.at[b, pl.ds(base + c * _CH, _CH)], sem).wait()

    start_pe(0, pe_bufs[0], pe_sems[0])
    start_loads(0, x_sets[0], in_sems[0])

    def pair_body(p, _):
        for kc in range(2):
            c = p * 2 + kc
            pe_cur = pe_bufs[kc]
            cur = x_sets[kc]
            nxt = x_sets[1 - kc]

            @pl.when(jnp.logical_and(c >= 1, c + 1 < n_chunks))
            def _():
                wait_stores(c - 1, nxt, out_sems[1 - kc])

            @pl.when(c + 1 < n_chunks)
            def _():
                start_loads(c + 1, nxt, in_sems[1 - kc])

            wait_pe(c, pe_cur, pe_sems[kc])

            @pl.when(c + 1 < n_chunks)
            def _():
                start_pe(c + 1, pe_bufs[1 - kc], pe_sems[1 - kc])

            wait_loads(c, cur, in_sems[kc])

            def row_body(r, _):
                for j in range(vec_per_row):
                    sl = pl.ds(j * _LANES, _LANES)
                    pp = pe_cur[r, sl]
                    for b in range(batch):
                        xx = cur[b][r, sl]
                        cur[b][r, sl] = jnp.where(xx == 0.0, xx, xx + pp)
                return 0

            lax.fori_loop(0, _CH, row_body, 0)
            start_stores(c, cur, out_sems[kc])
        return 0

    lax.fori_loop(0, n_chunks // 2, pair_body, 0)

    wait_stores(n_chunks - 2, x_sets[(n_chunks - 2) % 2], out_sems[(n_chunks - 2) % 2])
    wait_stores(n_chunks - 1, x_sets[(n_chunks - 1) % 2], out_sems[(n_chunks - 1) % 2])


def kernel(x, pos_embed):
    batch, seq, dim = x.shape
    pe = pos_embed[:seq]
    mesh = plsc.VectorSubcoreMesh(core_axis_name="c", subcore_axis_name="s")
    k = functools.partial(
        pl.kernel,
        mesh=mesh,
        out_type=jax.ShapeDtypeStruct(x.shape, x.dtype),
        scratch_types=(
            [pltpu.VMEM((_CH, dim), jnp.float32) for _ in range(10)]
            + [pltpu.SemaphoreType.DMA for _ in range(6)]
        ),
    )(_sc_kernel_body)
    return k(x, pe)
